# baseline (speedup 1.0000x reference)
import functools

import jax
import jax.numpy as jnp
from jax import lax
from jax.experimental import pallas as pl
from jax.experimental.pallas import tpu as pltpu
from jax.experimental.pallas import tpu_sc as plsc

_B = 65536
_SIN = 64
_SOUT = 256
_L = 16
_NC = 2
_NS = 16
_NW = _NC * _NS
_ROWS_PER_W = _B // _NW
_CHUNK = 128
_NCHUNK = _ROWS_PER_W // _CHUNK


def _sc_body(x_hbm, w_hbm, no_hbm, out_hbm, no_v, w_v, wfull, inv, xbuf, obuf):
    wid = lax.axis_index("s") * _NC + lax.axis_index("c")
    base = wid * _ROWS_PER_W

    pltpu.sync_copy(no_hbm, no_v)
    pltpu.sync_copy(w_hbm, w_v)

    zf = jnp.zeros((_L,), jnp.float32)
    zi = jnp.zeros((_L,), jnp.int32)
    for k in range(_SOUT // _L):
        wfull[pl.ds(k * _L, _L)] = zf
        inv[pl.ds(k * _L, _L)] = zi
    for k in range(_SIN // _L):
        idx = no_v[pl.ds(k * _L, _L)]
        plsc.store_scatter(wfull, [idx], w_v[pl.ds(k * _L, _L)])
        plsc.store_scatter(inv, [idx], lax.iota(jnp.int32, _L) + k * _L)

    invs = [inv[pl.ds(j * _L, _L)] for j in range(_SOUT // _L)]
    wfs = [wfull[pl.ds(j * _L, _L)] for j in range(_SOUT // _L)]

    def chunk_body(t, carry):
        r0 = base + t * _CHUNK
        pltpu.sync_copy(x_hbm.at[pl.ds(r0, _CHUNK)], xbuf)

        def row_body(r, c2):
            rsplat = jnp.full((_L,), r, jnp.int32)
            for j in range(_SOUT // _L):
                g = plsc.load_gather(xbuf, [rsplat, invs[j]])
                obuf[r, pl.ds(j * _L, _L)] = g * wfs[j]
            return c2

        lax.fori_loop(0, _CHUNK, row_body, 0)
        pltpu.sync_copy(obuf, out_hbm.at[pl.ds(r0, _CHUNK)])
        return carry

    lax.fori_loop(0, _NCHUNK, chunk_body, 0)


def _make_call():
    return pl.kernel(
        _sc_body,
        out_type=jax.ShapeDtypeStruct((_B, _SOUT), jnp.float32),
        mesh=plsc.VectorSubcoreMesh(
            core_axis_name="c", subcore_axis_name="s", num_cores=_NC, num_subcores=_NS
        ),
        compiler_params=pltpu.CompilerParams(needs_layout_passes=False),
        scratch_types=[
            pltpu.VMEM((_SIN,), jnp.int32),
            pltpu.VMEM((_SIN,), jnp.float32),
            pltpu.VMEM((_SOUT,), jnp.float32),
            pltpu.VMEM((_SOUT,), jnp.int32),
            pltpu.VMEM((_CHUNK, _SIN), jnp.float32),
            pltpu.VMEM((_CHUNK, _SOUT), jnp.float32),
        ],
    )


@jax.jit
def kernel(x, weights, node_order):
    return _make_call()(x, weights, node_order)


# speedup vs baseline: 3.0909x; 3.0909x over previous
import jax
import jax.numpy as jnp
from jax import lax
from jax.experimental import pallas as pl
from jax.experimental.pallas import tpu as pltpu
from jax.experimental.pallas import tpu_sc as plsc

_B = 65536
_SIN = 64
_SOUT = 256
_L = 16
_NC = 2
_NS = 16
_NW = _NC * _NS
_ROWS_PER_W = _B // _NW
_CHUNK = 128
_NCHUNK = _ROWS_PER_W // _CHUNK
_UNROLL = 4


def _sc_body(x_hbm, w_hbm, no_hbm, out_hbm, no_v, w_v, xbuf, obuf,
             isem0, isem1, osem0, osem1):
    wid = lax.axis_index("s") * _NC + lax.axis_index("c")
    base = wid * _ROWS_PER_W

    pltpu.sync_copy(no_hbm, no_v)
    pltpu.sync_copy(w_hbm, w_v)
    nov = [no_v[pl.ds(k * _L, _L)] for k in range(_SIN // _L)]
    wv = [w_v[pl.ds(k * _L, _L)] for k in range(_SIN // _L)]

    zf = jnp.zeros((_L,), jnp.float32)

    def zero_body(r, c):
        for b in range(2):
            for j in range(_SOUT // _L):
                obuf[b, r, pl.ds(j * _L, _L)] = zf
        return c

    lax.fori_loop(0, _CHUNK, zero_body, 0)

    isems = [isem0, isem1]
    osems = [osem0, osem1]

    for b in range(2):
        pltpu.async_copy(
            x_hbm.at[pl.ds(base + b * _CHUNK, _CHUNK)], xbuf.at[b], isems[b]
        )

    def outer(t, carry):
        for b in range(2):
            chunk = 2 * t + b
            r0 = base + chunk * _CHUNK
            pltpu.make_async_copy(
                x_hbm.at[pl.ds(r0, _CHUNK)], xbuf.at[b], isems[b]
            ).wait()

            @pl.when(t > 0)
            def _wait_out():
                pltpu.make_async_copy(
                    obuf.at[b], out_hbm.at[pl.ds(r0, _CHUNK)], osems[b]
                ).wait()

            def row_body(i, cc):
                r = i * _UNROLL
                for u in range(_UNROLL):
                    rs = jnp.full((_L,), r + u, jnp.int32)
                    for k in range(_SIN // _L):
                        v = xbuf[b, r + u, pl.ds(k * _L, _L)] * wv[k]
                        plsc.store_scatter(obuf.at[b], [rs, nov[k]], v)
                return cc

            lax.fori_loop(0, _CHUNK // _UNROLL, row_body, 0)

            pltpu.async_copy(obuf.at[b], out_hbm.at[pl.ds(r0, _CHUNK)], osems[b])

            @pl.when(chunk + 2 < _NCHUNK)
            def _next_in():
                pltpu.async_copy(
                    x_hbm.at[pl.ds(r0 + 2 * _CHUNK, _CHUNK)], xbuf.at[b], isems[b]
                )

        return carry

    lax.fori_loop(0, _NCHUNK // 2, outer, 0)

    for b in range(2):
        pltpu.make_async_copy(
            obuf.at[b], out_hbm.at[pl.ds(base, _CHUNK)], osems[b]
        ).wait()


def _make_call():
    return pl.kernel(
        _sc_body,
        out_type=jax.ShapeDtypeStruct((_B, _SOUT), jnp.float32),
        mesh=plsc.VectorSubcoreMesh(
            core_axis_name="c", subcore_axis_name="s", num_cores=_NC, num_subcores=_NS
        ),
        compiler_params=pltpu.CompilerParams(needs_layout_passes=False),
        scratch_types=[
            pltpu.VMEM((_SIN,), jnp.int32),
            pltpu.VMEM((_SIN,), jnp.float32),
            pltpu.VMEM((2, _CHUNK, _SIN), jnp.float32),
            pltpu.VMEM((2, _CHUNK, _SOUT), jnp.float32),
            pltpu.SemaphoreType.DMA,
            pltpu.SemaphoreType.DMA,
            pltpu.SemaphoreType.DMA,
            pltpu.SemaphoreType.DMA,
        ],
    )


@jax.jit
def kernel(x, weights, node_order):
    return _make_call()(x, weights, node_order)
